# baseline (speedup 1.0000x reference)
import functools

import numpy as np
import jax
import jax.numpy as jnp
from jax.experimental import pallas as pl
from jax.experimental.pallas import tpu as pltpu

_UNITS = 128
_MAX_N = 4
_MAX_L = 4
_CUTOFF = 5.0
_TB_CUTOFF = 4.0
_NUM_BLOCKS = 4

_ROOTS = np.array([
    [3.14159265, 6.28318531, 9.42477796, 12.56637061],
    [4.49340946, 7.72525184, 10.90412166, 14.06619391],
    [5.76345920, 9.09501133, 12.32294097, 15.51460301],
    [6.98793200, 10.41711855, 13.69802315, 16.92362129]])


def _swish(x):
    return x * jax.nn.sigmoid(x)


def _dot(a, b):
    return jax.lax.dot_general(a, b, (((1,), (0,)), ((), ())),
                               preferred_element_type=jnp.float32)


def _contrib_body(gjl_ref, gate_ref, ct_ref, feij_ref, out_ref):
    base = gjl_ref[...] * gate_ref[...]
    ct = ct_ref[...]
    feij = feij_ref[...]
    p2 = 0.5 * (3.0 * ct ** 2 - 1.0)
    p3 = 0.5 * (5.0 * ct ** 3 - 3.0 * ct)
    out_ref[...] = jnp.concatenate([
        base[:, 0:4],
        base[:, 4:8] * ct,
        base[:, 8:12] * p2,
        base[:, 12:16] * p3,
    ], axis=1) * feij


def _contrib_call(gjl, gateg, ct_col, feij_col):
    rows = gjl.shape[0]
    rb = 2048
    grid = rows // rb
    return pl.pallas_call(
        _contrib_body,
        grid=(grid,),
        in_specs=[
            pl.BlockSpec((rb, 16), lambda i: (i, 0)),
            pl.BlockSpec((rb, 16), lambda i: (i, 0)),
            pl.BlockSpec((rb, 1), lambda i: (i, 0)),
            pl.BlockSpec((rb, 1), lambda i: (i, 0)),
        ],
        out_specs=pl.BlockSpec((rb, 16), lambda i: (i, 0)),
        out_shape=jax.ShapeDtypeStruct((rows, 16), jnp.float32),
    )(gjl, gateg, ct_col, feij_col)


def _edge_body(vi_ref, vj_ref, e_ref, m_ref, e0_ref,
               wt1, bt1, wt2, bt2, we1, be1, we2, be2, wre,
               wn1, bn1, wn2, bn2, wrn,
               eout_ref, msg_ref):
    vi = vi_ref[...]
    vj = vj_ref[...]
    e = e_ref[...]
    m = m_ref[...]
    e0 = e0_ref[...]

    e = e + _swish(_dot(m, wt1[...]) + bt1[...]) * jax.nn.sigmoid(
        _dot(m, wt2[...]) + bt2[...])

    a1 = we1[...]
    a2 = we2[...]
    u = _UNITS
    g1 = _dot(vi, a1[0:u]) + _dot(vj, a1[u:2 * u]) + _dot(e, a1[2 * u:3 * u]) + be1[...]
    s1 = _dot(vi, a2[0:u]) + _dot(vj, a2[u:2 * u]) + _dot(e, a2[2 * u:3 * u]) + be2[...]
    e = e + _swish(g1) * jax.nn.sigmoid(s1) * _dot(e0, wre[...])

    b1 = wn1[...]
    b2 = wn2[...]
    g2 = _dot(vi, b1[0:u]) + _dot(vj, b1[u:2 * u]) + _dot(e, b1[2 * u:3 * u]) + bn1[...]
    s2 = _dot(vi, b2[0:u]) + _dot(vj, b2[u:2 * u]) + _dot(e, b2[2 * u:3 * u]) + bn2[...]
    eout_ref[...] = e
    msg_ref[...] = _swish(g2) * jax.nn.sigmoid(s2) * _dot(e0, wrn[...])


def _edge_call(vi, vj, e, m, e0, wts):
    n_edges = vi.shape[0]
    eb = 640
    grid = n_edges // eb
    u = _UNITS
    full = lambda shape: pl.BlockSpec(shape, lambda i: tuple(0 for _ in shape))
    w_specs = [
        full((16, u)), full((1, u)), full((16, u)), full((1, u)),
        full((3 * u, u)), full((1, u)), full((3 * u, u)), full((1, u)),
        full((_MAX_N, u)),
        full((3 * u, u)), full((1, u)), full((3 * u, u)), full((1, u)),
        full((_MAX_N, u)),
    ]
    return pl.pallas_call(
        _edge_body,
        grid=(grid,),
        in_specs=[
            pl.BlockSpec((eb, u), lambda i: (i, 0)),
            pl.BlockSpec((eb, u), lambda i: (i, 0)),
            pl.BlockSpec((eb, u), lambda i: (i, 0)),
            pl.BlockSpec((eb, 16), lambda i: (i, 0)),
            pl.BlockSpec((eb, _MAX_N), lambda i: (i, 0)),
        ] + w_specs,
        out_specs=[
            pl.BlockSpec((eb, u), lambda i: (i, 0)),
            pl.BlockSpec((eb, u), lambda i: (i, 0)),
        ],
        out_shape=[
            jax.ShapeDtypeStruct((n_edges, u), jnp.float32),
            jax.ShapeDtypeStruct((n_edges, u), jnp.float32),
        ],
    )(vi, vj, e, m, e0, *wts)


def _readout_body(v_ref, wf1, bf1, wf2, bf2, wf3, bf3,
                  wh1, bh1, wh2, bh2, wh3, bh3, out_ref):
    v = v_ref[...]
    h = _swish(_dot(v, wf1[...]) + bf1[...])
    h = _swish(_dot(h, wf2[...]) + bf2[...])
    main = _dot(h, wf3[...]) + bf3[...]
    g = _swish(_dot(v, wh1[...]) + bh1[...])
    g = _swish(_dot(g, wh2[...]) + bh2[...])
    g = jax.nn.sigmoid(_dot(g, wh3[...]) + bh3[...])
    out_ref[...] = main * g


def _readout_call(v, wts):
    n = v.shape[0]
    nb = 1024
    grid = n // nb
    u = _UNITS
    full = lambda shape: pl.BlockSpec(shape, lambda i: tuple(0 for _ in shape))
    w_specs = [
        full((u, u)), full((1, u)), full((u, u)), full((1, u)),
        full((u, 1)), full((1, 1)),
        full((u, u)), full((1, u)), full((u, u)), full((1, u)),
        full((u, 1)), full((1, 1)),
    ]
    return pl.pallas_call(
        _readout_body,
        grid=(grid,),
        in_specs=[pl.BlockSpec((nb, u), lambda i: (i, 0))] + w_specs,
        out_specs=pl.BlockSpec((nb, 1), lambda i: (i, 0)),
        out_shape=jax.ShapeDtypeStruct((n, 1), jnp.float32),
    )(v, *wts)


def _smooth_bessel(r):
    fs = []
    for i in range(_MAX_N):
        c = ((-1.0) ** i) * np.sqrt(2.0) * np.pi / _CUTOFF ** 1.5 * (i + 1) * (
            i + 2) / np.sqrt((i + 1) ** 2 + (i + 2) ** 2)
        x1 = (i + 1) * np.pi * r / _CUTOFF
        x2 = (i + 2) * np.pi * r / _CUTOFF
        fs.append(c * (jnp.sin(x1) / x1 + jnp.sin(x2) / x2))
    en = [i ** 2 * (i + 2) ** 2 / (4.0 * (i + 1) ** 4 + 1.0) for i in range(_MAX_N)]
    d = [1.0]
    g = [fs[0]]
    for i in range(1, _MAX_N):
        dn = 1.0 - en[i] / d[i - 1]
        d.append(dn)
        g.append((fs[i] + np.sqrt(en[i] / d[i - 1]) * g[i - 1]) / np.sqrt(dn))
    return jnp.stack(g, axis=1)


def kernel(atom_pos, cell, pbc_offsets, atom_attr, edge_index, three_body_indices,
           batch, params):
    p = params
    src = edge_index[0]
    dst = edge_index[1]
    n_atoms = atom_pos.shape[0]
    n_edges = src.shape[0]
    n_triples = three_body_indices.shape[0]
    num_graphs = cell.shape[0]

    offset = jnp.einsum("bi,bij->bj", pbc_offsets, cell[batch[src]])
    edge_vec = atom_pos[src] - (atom_pos[dst] + offset)
    edge_len = jnp.sqrt(jnp.sum(edge_vec * edge_vec, axis=1)) + 1e-9

    ij = three_body_indices[:, 0]
    ik = three_body_indices[:, 1]
    vij = edge_vec[ij]
    vik = edge_vec[ik]
    rij = edge_len[ij]
    rik = edge_len[ik]
    ct = jnp.clip(jnp.sum(vij * vik, axis=1) / (rij * rik), -1.0 + 1e-7, 1.0 - 1e-7)
    ct = jnp.cos(jnp.arccos(ct))
    k_atom = dst[ik]

    jl_cols = []
    for l in range(_MAX_L):
        for n in range(_MAX_N):
            x = _ROOTS[l, n] * edge_len / _CUTOFF + 1e-9
            s = jnp.sin(x)
            c = jnp.cos(x)
            if l == 0:
                jl = s / x
            elif l == 1:
                jl = s / x ** 2 - c / x
            elif l == 2:
                jl = (3.0 / x ** 3 - 1.0 / x) * s - 3.0 * c / x ** 2
            else:
                jl = (15.0 / x ** 4 - 6.0 / x ** 2) * s - (15.0 / x ** 3 - 1.0 / x) * c
            jl_cols.append(jl)
    jl_tab = jnp.stack(jl_cols, axis=1)

    u_fc = edge_len / _TB_CUTOFF
    fe = jnp.where(edge_len < _TB_CUTOFF,
                   1.0 - 6.0 * u_fc ** 5 + 15.0 * u_fc ** 4 - 10.0 * u_fc ** 3, 0.0)
    gtab = jl_tab * fe[:, None]

    tpad = ((n_triples + 2047) // 2048) * 2048
    tp = tpad - n_triples
    gjl = jnp.concatenate([gtab[ik], jnp.zeros((tp, 16), jnp.float32)])
    feij_col = jnp.concatenate([fe[ij], jnp.zeros((tp,), jnp.float32)])[:, None]
    ct_col = jnp.concatenate([ct, jnp.zeros((tp,), jnp.float32)])[:, None]

    z = atom_attr[:, 0]
    v = p["atom_emb"][z]
    e0 = _smooth_bessel(edge_len)
    e = e0 @ p["edge_enc"]

    epad = ((n_edges + 639) // 640) * 640
    src_p = jnp.concatenate([src, jnp.zeros((epad - n_edges,), src.dtype)])
    dst_p = jnp.concatenate([dst, jnp.zeros((epad - n_edges,), dst.dtype)])
    e0 = jnp.concatenate([e0, jnp.zeros((epad - n_edges, _MAX_N), jnp.float32)])
    e = jnp.concatenate([e, jnp.zeros((epad - n_edges, _UNITS), jnp.float32)])

    for b in range(_NUM_BLOCKS):
        pre = "b%d_" % b
        gate = jax.nn.sigmoid(v @ p[pre + "Wg"] + p[pre + "bg"])
        gateg = jnp.concatenate([gate[k_atom], jnp.zeros((tp, 16), jnp.float32)])
        contrib = _contrib_call(gjl, gateg, ct_col, feij_col)
        m = jax.ops.segment_sum(contrib[:n_triples], ij, num_segments=epad)
        vi = v[src_p]
        vj = v[dst_p]
        wts = [p[pre + "Wt1"], p[pre + "bt1"][None], p[pre + "Wt2"], p[pre + "bt2"][None],
               p[pre + "We1"], p[pre + "be1"][None], p[pre + "We2"], p[pre + "be2"][None],
               p[pre + "Wre"],
               p[pre + "Wn1"], p[pre + "bn1"][None], p[pre + "Wn2"], p[pre + "bn2"][None],
               p[pre + "Wrn"]]
        e, msg = _edge_call(vi, vj, e, m, e0, wts)
        v = v + jax.ops.segment_sum(msg[:n_edges], src, num_segments=n_atoms)

    apad = ((n_atoms + 1023) // 1024) * 1024
    vp = jnp.concatenate([v, jnp.zeros((apad - n_atoms, _UNITS), jnp.float32)])
    r_wts = [p["Wf1"], p["bf1"][None], p["Wf2"], p["bf2"][None], p["Wf3"], p["bf3"][None],
             p["Wh1"], p["bh1"][None], p["Wh2"], p["bh2"][None], p["Wh3"], p["bh3"][None]]
    ei_raw = _readout_call(vp, r_wts)[:n_atoms, 0]
    ei = ei_raw * p["gamma"][z] + p["beta"][z]
    return jax.ops.segment_sum(ei, batch, num_segments=num_graphs)


# speedup vs baseline: 1.9153x; 1.9153x over previous
import functools

import numpy as np
import jax
import jax.numpy as jnp
from jax.experimental import pallas as pl
from jax.experimental.pallas import tpu as pltpu

_UNITS = 128
_MAX_N = 4
_MAX_L = 4
_CUTOFF = 5.0
_TB_CUTOFF = 4.0
_NUM_BLOCKS = 4

_ROOTS = np.array([
    [3.14159265, 6.28318531, 9.42477796, 12.56637061],
    [4.49340946, 7.72525184, 10.90412166, 14.06619391],
    [5.76345920, 9.09501133, 12.32294097, 15.51460301],
    [6.98793200, 10.41711855, 13.69802315, 16.92362129]])


def _swish(x):
    return x * jax.nn.sigmoid(x)


def _dot(a, b):
    return jax.lax.dot_general(a, b, (((1,), (0,)), ((), ())),
                               preferred_element_type=jnp.float32)


def _contrib_body(gjl_ref, gate_ref, ct_ref, feij_ref, out_ref):
    base = gjl_ref[...] * gate_ref[...]
    ct = ct_ref[...]
    feij = feij_ref[...]
    p2 = 0.5 * (3.0 * ct ** 2 - 1.0)
    p3 = 0.5 * (5.0 * ct ** 3 - 3.0 * ct)
    out_ref[...] = jnp.concatenate([
        base[:, 0:4],
        base[:, 4:8] * ct,
        base[:, 8:12] * p2,
        base[:, 12:16] * p3,
    ], axis=1) * feij


def _contrib_call(gjl, gateg, ct_col, feij_col):
    rows = gjl.shape[0]
    rb = 2048
    grid = rows // rb
    return pl.pallas_call(
        _contrib_body,
        grid=(grid,),
        in_specs=[
            pl.BlockSpec((rb, 16), lambda i: (i, 0)),
            pl.BlockSpec((rb, 16), lambda i: (i, 0)),
            pl.BlockSpec((rb, 1), lambda i: (i, 0)),
            pl.BlockSpec((rb, 1), lambda i: (i, 0)),
        ],
        out_specs=pl.BlockSpec((rb, 16), lambda i: (i, 0)),
        out_shape=jax.ShapeDtypeStruct((rows, 16), jnp.float32),
    )(gjl, gateg, ct_col, feij_col)


def _edge_body(vi_ref, vj_ref, e_ref, m_ref, e0_ref,
               wt1, bt1, wt2, bt2, we1, be1, we2, be2, wre,
               wn1, bn1, wn2, bn2, wrn,
               eout_ref, msg_ref):
    vi = vi_ref[...]
    vj = vj_ref[...]
    e = e_ref[...]
    m = m_ref[...]
    e0 = e0_ref[...]

    e = e + _swish(_dot(m, wt1[...]) + bt1[...]) * jax.nn.sigmoid(
        _dot(m, wt2[...]) + bt2[...])

    a1 = we1[...]
    a2 = we2[...]
    u = _UNITS
    g1 = _dot(vi, a1[0:u]) + _dot(vj, a1[u:2 * u]) + _dot(e, a1[2 * u:3 * u]) + be1[...]
    s1 = _dot(vi, a2[0:u]) + _dot(vj, a2[u:2 * u]) + _dot(e, a2[2 * u:3 * u]) + be2[...]
    e = e + _swish(g1) * jax.nn.sigmoid(s1) * _dot(e0, wre[...])

    b1 = wn1[...]
    b2 = wn2[...]
    g2 = _dot(vi, b1[0:u]) + _dot(vj, b1[u:2 * u]) + _dot(e, b1[2 * u:3 * u]) + bn1[...]
    s2 = _dot(vi, b2[0:u]) + _dot(vj, b2[u:2 * u]) + _dot(e, b2[2 * u:3 * u]) + bn2[...]
    eout_ref[...] = e
    msg_ref[...] = _swish(g2) * jax.nn.sigmoid(s2) * _dot(e0, wrn[...])


def _edge_call(vi, vj, e, m, e0, wts):
    n_edges = vi.shape[0]
    eb = 640
    grid = n_edges // eb
    u = _UNITS
    full = lambda shape: pl.BlockSpec(shape, lambda i: tuple(0 for _ in shape))
    w_specs = [
        full((16, u)), full((1, u)), full((16, u)), full((1, u)),
        full((3 * u, u)), full((1, u)), full((3 * u, u)), full((1, u)),
        full((_MAX_N, u)),
        full((3 * u, u)), full((1, u)), full((3 * u, u)), full((1, u)),
        full((_MAX_N, u)),
    ]
    return pl.pallas_call(
        _edge_body,
        grid=(grid,),
        in_specs=[
            pl.BlockSpec((eb, u), lambda i: (i, 0)),
            pl.BlockSpec((eb, u), lambda i: (i, 0)),
            pl.BlockSpec((eb, u), lambda i: (i, 0)),
            pl.BlockSpec((eb, 16), lambda i: (i, 0)),
            pl.BlockSpec((eb, _MAX_N), lambda i: (i, 0)),
        ] + w_specs,
        out_specs=[
            pl.BlockSpec((eb, u), lambda i: (i, 0)),
            pl.BlockSpec((eb, u), lambda i: (i, 0)),
        ],
        out_shape=[
            jax.ShapeDtypeStruct((n_edges, u), jnp.float32),
            jax.ShapeDtypeStruct((n_edges, u), jnp.float32),
        ],
    )(vi, vj, e, m, e0, *wts)


def _readout_body(v_ref, wf1, bf1, wf2, bf2, wf3, bf3,
                  wh1, bh1, wh2, bh2, wh3, bh3, out_ref):
    v = v_ref[...]
    h = _swish(_dot(v, wf1[...]) + bf1[...])
    h = _swish(_dot(h, wf2[...]) + bf2[...])
    main = _dot(h, wf3[...]) + bf3[...]
    g = _swish(_dot(v, wh1[...]) + bh1[...])
    g = _swish(_dot(g, wh2[...]) + bh2[...])
    g = jax.nn.sigmoid(_dot(g, wh3[...]) + bh3[...])
    out_ref[...] = main * g


def _readout_call(v, wts):
    n = v.shape[0]
    nb = 1024
    grid = n // nb
    u = _UNITS
    full = lambda shape: pl.BlockSpec(shape, lambda i: tuple(0 for _ in shape))
    w_specs = [
        full((u, u)), full((1, u)), full((u, u)), full((1, u)),
        full((u, 1)), full((1, 1)),
        full((u, u)), full((1, u)), full((u, u)), full((1, u)),
        full((u, 1)), full((1, 1)),
    ]
    return pl.pallas_call(
        _readout_body,
        grid=(grid,),
        in_specs=[pl.BlockSpec((nb, u), lambda i: (i, 0))] + w_specs,
        out_specs=pl.BlockSpec((nb, 1), lambda i: (i, 0)),
        out_shape=jax.ShapeDtypeStruct((n, 1), jnp.float32),
    )(v, *wts)


def _smooth_bessel(r):
    fs = []
    for i in range(_MAX_N):
        c = ((-1.0) ** i) * np.sqrt(2.0) * np.pi / _CUTOFF ** 1.5 * (i + 1) * (
            i + 2) / np.sqrt((i + 1) ** 2 + (i + 2) ** 2)
        x1 = (i + 1) * np.pi * r / _CUTOFF
        x2 = (i + 2) * np.pi * r / _CUTOFF
        fs.append(c * (jnp.sin(x1) / x1 + jnp.sin(x2) / x2))
    en = [i ** 2 * (i + 2) ** 2 / (4.0 * (i + 1) ** 4 + 1.0) for i in range(_MAX_N)]
    d = [1.0]
    g = [fs[0]]
    for i in range(1, _MAX_N):
        dn = 1.0 - en[i] / d[i - 1]
        d.append(dn)
        g.append((fs[i] + np.sqrt(en[i] / d[i - 1]) * g[i - 1]) / np.sqrt(dn))
    return jnp.stack(g, axis=1)


def kernel(atom_pos, cell, pbc_offsets, atom_attr, edge_index, three_body_indices,
           batch, params):
    p = params
    src = edge_index[0]
    dst = edge_index[1]
    n_atoms = atom_pos.shape[0]
    n_edges = src.shape[0]
    n_triples = three_body_indices.shape[0]
    num_graphs = cell.shape[0]

    offset = jnp.einsum("bi,bij->bj", pbc_offsets, cell[batch[src]])
    edge_vec = atom_pos[src] - (atom_pos[dst] + offset)
    edge_len = jnp.sqrt(jnp.sum(edge_vec * edge_vec, axis=1)) + 1e-9

    ij = three_body_indices[:, 0]
    ik = three_body_indices[:, 1]

    jl_cols = []
    for l in range(_MAX_L):
        for n in range(_MAX_N):
            x = _ROOTS[l, n] * edge_len / _CUTOFF + 1e-9
            s = jnp.sin(x)
            c = jnp.cos(x)
            if l == 0:
                jl = s / x
            elif l == 1:
                jl = s / x ** 2 - c / x
            elif l == 2:
                jl = (3.0 / x ** 3 - 1.0 / x) * s - 3.0 * c / x ** 2
            else:
                jl = (15.0 / x ** 4 - 6.0 / x ** 2) * s - (15.0 / x ** 3 - 1.0 / x) * c
            jl_cols.append(jl)
    jl_tab = jnp.stack(jl_cols, axis=1)

    u_fc = edge_len / _TB_CUTOFF
    fe = jnp.where(edge_len < _TB_CUTOFF,
                   1.0 - 6.0 * u_fc ** 5 + 15.0 * u_fc ** 4 - 10.0 * u_fc ** 3, 0.0)
    gtab = jl_tab * fe[:, None]

    unit = edge_vec / edge_len[:, None]
    t_ij = jnp.concatenate([unit, fe[:, None]], axis=1)
    t_ik = jnp.concatenate([unit, dst.astype(jnp.float32)[:, None], gtab], axis=1)
    g_ij = t_ij[ij]
    g_ik = t_ik[ik]
    ct = jnp.clip(jnp.sum(g_ij[:, :3] * g_ik[:, :3], axis=1), -1.0 + 1e-7, 1.0 - 1e-7)
    ct = jnp.cos(jnp.arccos(ct))
    k_atom = g_ik[:, 3].astype(jnp.int32)

    tpad = ((n_triples + 2047) // 2048) * 2048
    tp = tpad - n_triples
    gjl = jnp.concatenate([g_ik[:, 4:], jnp.zeros((tp, 16), jnp.float32)])
    feij_col = jnp.concatenate([g_ij[:, 3], jnp.zeros((tp,), jnp.float32)])[:, None]
    ct_col = jnp.concatenate([ct, jnp.zeros((tp,), jnp.float32)])[:, None]

    z = atom_attr[:, 0]
    v = p["atom_emb"][z]
    e0 = _smooth_bessel(edge_len)
    e = e0 @ p["edge_enc"]

    epad = ((n_edges + 639) // 640) * 640
    src_p = jnp.concatenate([src, jnp.zeros((epad - n_edges,), src.dtype)])
    dst_p = jnp.concatenate([dst, jnp.zeros((epad - n_edges,), dst.dtype)])
    e0 = jnp.concatenate([e0, jnp.zeros((epad - n_edges, _MAX_N), jnp.float32)])
    e = jnp.concatenate([e, jnp.zeros((epad - n_edges, _UNITS), jnp.float32)])

    for b in range(_NUM_BLOCKS):
        pre = "b%d_" % b
        gate = jax.nn.sigmoid(v @ p[pre + "Wg"] + p[pre + "bg"])
        gateg = jnp.concatenate([gate[k_atom], jnp.zeros((tp, 16), jnp.float32)])
        contrib = _contrib_call(gjl, gateg, ct_col, feij_col)
        m = jax.ops.segment_sum(contrib[:n_triples], ij, num_segments=epad)
        vi = v[src_p]
        vj = v[dst_p]
        wts = [p[pre + "Wt1"], p[pre + "bt1"][None], p[pre + "Wt2"], p[pre + "bt2"][None],
               p[pre + "We1"], p[pre + "be1"][None], p[pre + "We2"], p[pre + "be2"][None],
               p[pre + "Wre"],
               p[pre + "Wn1"], p[pre + "bn1"][None], p[pre + "Wn2"], p[pre + "bn2"][None],
               p[pre + "Wrn"]]
        e, msg = _edge_call(vi, vj, e, m, e0, wts)
        v = v + jax.ops.segment_sum(msg[:n_edges], src, num_segments=n_atoms)

    apad = ((n_atoms + 1023) // 1024) * 1024
    vp = jnp.concatenate([v, jnp.zeros((apad - n_atoms, _UNITS), jnp.float32)])
    r_wts = [p["Wf1"], p["bf1"][None], p["Wf2"], p["bf2"][None], p["Wf3"], p["bf3"][None],
             p["Wh1"], p["bh1"][None], p["Wh2"], p["bh2"][None], p["Wh3"], p["bh3"][None]]
    ei_raw = _readout_call(vp, r_wts)[:n_atoms, 0]
    ei = ei_raw * p["gamma"][z] + p["beta"][z]
    return jax.ops.segment_sum(ei, batch, num_segments=num_graphs)
